# baseline (device time: 83226 ns/iter reference)
import jax
import jax.numpy as jnp
from jax import lax
from jax.experimental import pallas as pl
from jax.experimental.pallas import tpu as pltpu

N_DEV = 4


def kernel(A, B):
    m, k = A.shape
    k2, n = B.shape

    def body(a_ref, b_ref, out_ref, comm_ref, send_sems, recv_sems):
        my = lax.axis_index("i")
        left = (my - 1) % N_DEV
        right = (my + 1) % N_DEV

        barrier_sem = pltpu.get_barrier_semaphore()
        for nbr in (left, right):
            pl.semaphore_signal(
                barrier_sem, inc=1,
                device_id=(nbr,), device_id_type=pl.DeviceIdType.MESH,
            )
        pl.semaphore_wait(barrier_sem, 2)

        partial = jnp.dot(
            a_ref[:, :].astype(jnp.bfloat16),
            b_ref[:, :].astype(jnp.bfloat16),
            preferred_element_type=jnp.float32,
        )
        out_ref[:, :] = partial
        comm_ref[0, :, :] = partial.astype(jnp.bfloat16)

        for h in range(N_DEV - 1):
            rdma = pltpu.make_async_remote_copy(
                src_ref=comm_ref.at[h],
                dst_ref=comm_ref.at[h + 1],
                send_sem=send_sems.at[h],
                recv_sem=recv_sems.at[h],
                device_id=(right,),
                device_id_type=pl.DeviceIdType.MESH,
            )
            rdma.start()
            rdma.wait()
            out_ref[:, :] += comm_ref[h + 1, :, :].astype(jnp.float32)

    return pl.pallas_call(
        body,
        out_shape=jax.ShapeDtypeStruct((m, n), jnp.float32),
        in_specs=[
            pl.BlockSpec(memory_space=pltpu.VMEM),
            pl.BlockSpec(memory_space=pltpu.VMEM),
        ],
        out_specs=pl.BlockSpec(memory_space=pltpu.VMEM),
        scratch_shapes=[
            pltpu.VMEM((N_DEV, m, n), jnp.bfloat16),
            pltpu.SemaphoreType.DMA((N_DEV - 1,)),
            pltpu.SemaphoreType.DMA((N_DEV - 1,)),
        ],
        compiler_params=pltpu.CompilerParams(collective_id=0),
    )(A, B)


# device time: 36308 ns/iter; 2.2922x vs baseline; 2.2922x over previous
import jax
import jax.numpy as jnp
from jax import lax
from jax.experimental import pallas as pl
from jax.experimental.pallas import tpu as pltpu

N_DEV = 4


def kernel(A, B):
    m, k = A.shape
    k2, n = B.shape
    seg = m // N_DEV

    def body(a_ref, b_ref, out_ref, pbuf, rbuf, red_buf, bbuf,
             send_r, recv_r, send_b, recv_b):
        my = lax.axis_index("i")

        barrier_sem = pltpu.get_barrier_semaphore()
        for j in range(1, N_DEV):
            pl.semaphore_signal(
                barrier_sem, inc=1,
                device_id=((my + j) % N_DEV,),
                device_id_type=pl.DeviceIdType.MESH,
            )
        pl.semaphore_wait(barrier_sem, N_DEV - 1)

        partial = jnp.dot(
            a_ref[:, :].astype(jnp.bfloat16),
            b_ref[:, :].astype(jnp.bfloat16),
            preferred_element_type=jnp.float32,
        )
        pbuf[:, :] = partial.astype(jnp.bfloat16)
        out_ref[:, :] = partial

        rdmas_r = []
        for j in range(N_DEV - 1):
            t = (my + 1 + j) % N_DEV
            r = pltpu.make_async_remote_copy(
                src_ref=pbuf.at[pl.ds(t * seg, seg), :],
                dst_ref=rbuf.at[j],
                send_sem=send_r.at[j],
                recv_sem=recv_r.at[j],
                device_id=(t,),
                device_id_type=pl.DeviceIdType.MESH,
            )
            r.start()
            rdmas_r.append(r)

        acc = out_ref[pl.ds(my * seg, seg), :]
        for j in range(N_DEV - 1):
            rdmas_r[j].wait_recv()
            acc = acc + rbuf[j, :, :].astype(jnp.float32)
        out_ref[pl.ds(my * seg, seg), :] = acc
        red_buf[:, :] = acc.astype(jnp.bfloat16)

        rdmas_b = []
        for j in range(N_DEV - 1):
            t = (my + 1 + j) % N_DEV
            r = pltpu.make_async_remote_copy(
                src_ref=red_buf,
                dst_ref=bbuf.at[j],
                send_sem=send_b.at[j],
                recv_sem=recv_b.at[j],
                device_id=(t,),
                device_id_type=pl.DeviceIdType.MESH,
            )
            r.start()
            rdmas_b.append(r)

        for j in range(N_DEV - 1):
            rdmas_b[j].wait_recv()
            origin = (my - 1 - j) % N_DEV
            out_ref[pl.ds(origin * seg, seg), :] = (
                bbuf[j, :, :].astype(jnp.float32)
            )

        for j in range(N_DEV - 1):
            rdmas_r[j].wait_send()
            rdmas_b[j].wait_send()

    return pl.pallas_call(
        body,
        out_shape=jax.ShapeDtypeStruct((m, n), jnp.float32),
        in_specs=[
            pl.BlockSpec(memory_space=pltpu.VMEM),
            pl.BlockSpec(memory_space=pltpu.VMEM),
        ],
        out_specs=pl.BlockSpec(memory_space=pltpu.VMEM),
        scratch_shapes=[
            pltpu.VMEM((m, n), jnp.bfloat16),
            pltpu.VMEM((N_DEV - 1, seg, n), jnp.bfloat16),
            pltpu.VMEM((seg, n), jnp.bfloat16),
            pltpu.VMEM((N_DEV - 1, seg, n), jnp.bfloat16),
            pltpu.SemaphoreType.DMA((N_DEV - 1,)),
            pltpu.SemaphoreType.DMA((N_DEV - 1,)),
            pltpu.SemaphoreType.DMA((N_DEV - 1,)),
            pltpu.SemaphoreType.DMA((N_DEV - 1,)),
        ],
        compiler_params=pltpu.CompilerParams(collective_id=0),
    )(A, B)


# device time: 35575 ns/iter; 2.3395x vs baseline; 1.0206x over previous
import jax
import jax.numpy as jnp
from jax import lax
from jax.experimental import pallas as pl
from jax.experimental.pallas import tpu as pltpu

N_DEV = 4


def kernel(A, B):
    m, k = A.shape
    k2, n = B.shape
    seg = m // N_DEV

    def body(a_ref, b_ref, out_ref, pbuf, rbuf, red_buf, bbuf,
             send_r, recv_r, send_b, recv_b):
        my = lax.axis_index("i")

        barrier_sem = pltpu.get_barrier_semaphore()
        for j in range(1, N_DEV):
            pl.semaphore_signal(
                barrier_sem, inc=1,
                device_id=((my + j) % N_DEV,),
                device_id_type=pl.DeviceIdType.MESH,
            )
        pl.semaphore_wait(barrier_sem, N_DEV - 1)

        b_val = b_ref[:, :].astype(jnp.bfloat16)

        rdmas_r = []
        for j in range(N_DEV - 1):
            t = (my + 1 + j) % N_DEV
            seg_val = jnp.dot(
                a_ref[pl.ds(t * seg, seg), :].astype(jnp.bfloat16),
                b_val,
                preferred_element_type=jnp.float32,
            )
            pbuf[j, :, :] = seg_val.astype(jnp.bfloat16)
            r = pltpu.make_async_remote_copy(
                src_ref=pbuf.at[j],
                dst_ref=rbuf.at[j],
                send_sem=send_r.at[j],
                recv_sem=recv_r.at[j],
                device_id=(t,),
                device_id_type=pl.DeviceIdType.MESH,
            )
            r.start()
            rdmas_r.append(r)

        acc = jnp.dot(
            a_ref[pl.ds(my * seg, seg), :].astype(jnp.bfloat16),
            b_val,
            preferred_element_type=jnp.float32,
        )
        for j in range(N_DEV - 1):
            rdmas_r[j].wait_recv()
            acc = acc + rbuf[j, :, :].astype(jnp.float32)
        out_ref[pl.ds(my * seg, seg), :] = acc
        red_buf[:, :] = acc.astype(jnp.bfloat16)

        rdmas_b = []
        for j in range(N_DEV - 1):
            t = (my + 1 + j) % N_DEV
            r = pltpu.make_async_remote_copy(
                src_ref=red_buf,
                dst_ref=bbuf.at[j],
                send_sem=send_b.at[j],
                recv_sem=recv_b.at[j],
                device_id=(t,),
                device_id_type=pl.DeviceIdType.MESH,
            )
            r.start()
            rdmas_b.append(r)

        for j in range(N_DEV - 1):
            rdmas_b[j].wait_recv()
            origin = (my - 1 - j) % N_DEV
            out_ref[pl.ds(origin * seg, seg), :] = (
                bbuf[j, :, :].astype(jnp.float32)
            )

        for j in range(N_DEV - 1):
            rdmas_r[j].wait_send()
            rdmas_b[j].wait_send()

    return pl.pallas_call(
        body,
        out_shape=jax.ShapeDtypeStruct((m, n), jnp.float32),
        in_specs=[
            pl.BlockSpec(memory_space=pltpu.VMEM),
            pl.BlockSpec(memory_space=pltpu.VMEM),
        ],
        out_specs=pl.BlockSpec(memory_space=pltpu.VMEM),
        scratch_shapes=[
            pltpu.VMEM((N_DEV - 1, seg, n), jnp.bfloat16),
            pltpu.VMEM((N_DEV - 1, seg, n), jnp.bfloat16),
            pltpu.VMEM((seg, n), jnp.bfloat16),
            pltpu.VMEM((N_DEV - 1, seg, n), jnp.bfloat16),
            pltpu.SemaphoreType.DMA((N_DEV - 1,)),
            pltpu.SemaphoreType.DMA((N_DEV - 1,)),
            pltpu.SemaphoreType.DMA((N_DEV - 1,)),
            pltpu.SemaphoreType.DMA((N_DEV - 1,)),
        ],
        compiler_params=pltpu.CompilerParams(collective_id=0),
    )(A, B)


# device time: 33565 ns/iter; 2.4795x vs baseline; 1.0599x over previous
import jax
import jax.numpy as jnp
from jax import lax
from jax.experimental import pallas as pl
from jax.experimental.pallas import tpu as pltpu

N_DEV = 4
N_CHUNK = 2


def kernel(A, B):
    m, k = A.shape
    k2, n = B.shape
    seg = m // N_DEV
    nh = n // N_CHUNK

    def body(a_ref, b_ref, out_ref, pbuf, rbuf, red_buf, bbuf,
             send_r, recv_r, send_b, recv_b):
        my = lax.axis_index("i")

        barrier_sem = pltpu.get_barrier_semaphore()
        for j in range(1, N_DEV):
            pl.semaphore_signal(
                barrier_sem, inc=1,
                device_id=((my + j) % N_DEV,),
                device_id_type=pl.DeviceIdType.MESH,
            )
        pl.semaphore_wait(barrier_sem, N_DEV - 1)

        b_val = b_ref[:, :].astype(jnp.bfloat16)

        rdmas_r = []
        for j in range(N_DEV - 1):
            t = (my + 1 + j) % N_DEV
            seg_val = jnp.dot(
                a_ref[pl.ds(t * seg, seg), :].astype(jnp.bfloat16),
                b_val,
                preferred_element_type=jnp.float32,
            ).astype(jnp.bfloat16)
            per_chunk = []
            for c in range(N_CHUNK):
                pbuf[c, j, :, :] = seg_val[:, c * nh:(c + 1) * nh]
                r = pltpu.make_async_remote_copy(
                    src_ref=pbuf.at[c, j],
                    dst_ref=rbuf.at[c, j],
                    send_sem=send_r.at[c, j],
                    recv_sem=recv_r.at[c, j],
                    device_id=(t,),
                    device_id_type=pl.DeviceIdType.MESH,
                )
                r.start()
                per_chunk.append(r)
            rdmas_r.append(per_chunk)

        own = jnp.dot(
            a_ref[pl.ds(my * seg, seg), :].astype(jnp.bfloat16),
            b_val,
            preferred_element_type=jnp.float32,
        )

        rdmas_b = []
        for c in range(N_CHUNK):
            acc = own[:, c * nh:(c + 1) * nh]
            for j in range(N_DEV - 1):
                rdmas_r[j][c].wait_recv()
                acc = acc + rbuf[c, j, :, :].astype(jnp.float32)
            out_ref[pl.ds(my * seg, seg), pl.ds(c * nh, nh)] = acc
            red_buf[c, :, :] = acc.astype(jnp.bfloat16)
            per_chunk = []
            for j in range(N_DEV - 1):
                t = (my + 1 + j) % N_DEV
                r = pltpu.make_async_remote_copy(
                    src_ref=red_buf.at[c],
                    dst_ref=bbuf.at[c, j],
                    send_sem=send_b.at[c, j],
                    recv_sem=recv_b.at[c, j],
                    device_id=(t,),
                    device_id_type=pl.DeviceIdType.MESH,
                )
                r.start()
                per_chunk.append(r)
            rdmas_b.append(per_chunk)

        for c in range(N_CHUNK):
            for j in range(N_DEV - 1):
                rdmas_b[c][j].wait_recv()
                origin = (my - 1 - j) % N_DEV
                out_ref[pl.ds(origin * seg, seg), pl.ds(c * nh, nh)] = (
                    bbuf[c, j, :, :].astype(jnp.float32)
                )

        for c in range(N_CHUNK):
            for j in range(N_DEV - 1):
                rdmas_r[j][c].wait_send()
                rdmas_b[c][j].wait_send()

    return pl.pallas_call(
        body,
        out_shape=jax.ShapeDtypeStruct((m, n), jnp.float32),
        in_specs=[
            pl.BlockSpec(memory_space=pltpu.VMEM),
            pl.BlockSpec(memory_space=pltpu.VMEM),
        ],
        out_specs=pl.BlockSpec(memory_space=pltpu.VMEM),
        scratch_shapes=[
            pltpu.VMEM((N_CHUNK, N_DEV - 1, seg, nh), jnp.bfloat16),
            pltpu.VMEM((N_CHUNK, N_DEV - 1, seg, nh), jnp.bfloat16),
            pltpu.VMEM((N_CHUNK, seg, nh), jnp.bfloat16),
            pltpu.VMEM((N_CHUNK, N_DEV - 1, seg, nh), jnp.bfloat16),
            pltpu.SemaphoreType.DMA((N_CHUNK, N_DEV - 1)),
            pltpu.SemaphoreType.DMA((N_CHUNK, N_DEV - 1)),
            pltpu.SemaphoreType.DMA((N_CHUNK, N_DEV - 1)),
            pltpu.SemaphoreType.DMA((N_CHUNK, N_DEV - 1)),
        ],
        compiler_params=pltpu.CompilerParams(collective_id=0),
    )(A, B)
